# baseline (speedup 1.0000x reference)
import functools

import jax
import jax.numpy as jnp
from jax import lax
from jax.experimental import pallas as pl
from jax.experimental.pallas import tpu as pltpu
from jax.experimental.pallas import tpu_sc as plsc


def kernel(x, table):
    B, S, D = x.shape
    NC, NS = 2, 16
    NW = NC * NS
    SPW = S // NW
    CS = 2
    NCH = SPW // CS
    NITER = NCH // 2
    NBUF = 2 * B

    mesh = plsc.VectorSubcoreMesh(core_axis_name="c", subcore_axis_name="s")

    @functools.partial(
        pl.kernel,
        out_type=jax.ShapeDtypeStruct((B, S, D), jnp.float32),
        mesh=mesh,
        scratch_types=[
            pltpu.VMEM((2, CS, D), jnp.float32),
            pltpu.VMEM((NBUF, CS, D), jnp.float32),
            pltpu.SemaphoreType.DMA((2,)),
            pltpu.SemaphoreType.DMA((NBUF,)),
            pltpu.SemaphoreType.DMA((NBUF,)),
        ],
    )
    def sc_add(x_hbm, t_hbm, o_hbm, t_v, xo_v, t_sem, in_sem, out_sem):
        wid = lax.axis_index("s") * NC + lax.axis_index("c")
        s_base = wid * SPW

        def s0(c):
            return s_base + c * CS

        def fire_t(c, p):
            pltpu.async_copy(
                t_hbm.at[pl.ds(s0(c), CS), :], t_v.at[p], t_sem.at[p]
            )

        def wait_t(c, p):
            pltpu.make_async_copy(
                t_hbm.at[pl.ds(s0(c), CS), :], t_v.at[p], t_sem.at[p]
            ).wait()

        def fire_in(c, b, m):
            pltpu.async_copy(
                x_hbm.at[b, pl.ds(s0(c), CS), :], xo_v.at[m], in_sem.at[m]
            )

        def wait_in(c, b, m):
            pltpu.make_async_copy(
                x_hbm.at[b, pl.ds(s0(c), CS), :], xo_v.at[m], in_sem.at[m]
            ).wait()

        def fire_out(c, b, m):
            pltpu.async_copy(
                xo_v.at[m], o_hbm.at[b, pl.ds(s0(c), CS), :], out_sem.at[m]
            )

        def wait_out(c, b, m):
            pltpu.make_async_copy(
                xo_v.at[m], o_hbm.at[b, pl.ds(s0(c), CS), :], out_sem.at[m]
            ).wait()

        fire_t(0, 0)
        fire_t(1, 1)
        for p in range(2):
            for b in range(B):
                fire_in(p, b, p * B + b)

        @pl.loop(0, NITER)
        def _iter(cc):
            for p in range(2):
                c = 2 * cc + p
                wait_t(c, p)
                for b in range(B):
                    m = p * B + b
                    wait_in(c, b, m)
                    for r in range(CS):

                        @pl.loop(0, D // 16, unroll=8)
                        def _vec(i):
                            sl = pl.ds(i * 16, 16)
                            plsc.addupdate(xo_v.at[m, r, sl], t_v[p, r, sl])

                    fire_out(c, b, m)

                @pl.when(c + 2 < NCH)
                def _():
                    fire_t(c + 2, p)

                q = 1 - p
                if p == 0:
                    cond = cc > 0
                else:
                    cond = cc < NITER - 1

                @pl.when(cond)
                def _():
                    for b in range(B):
                        m = q * B + b
                        wait_out(c - 1, b, m)
                        fire_in(c + 1, b, m)

        for p in range(2):
            for b in range(B):
                wait_out(NCH - 2 + p, b, p * B + b)

    return sc_add(x, table)


# speedup vs baseline: 1.0029x; 1.0029x over previous
import functools

import jax
import jax.numpy as jnp
from jax import lax
from jax.experimental import pallas as pl
from jax.experimental.pallas import tpu as pltpu
from jax.experimental.pallas import tpu_sc as plsc


def kernel(x, table):
    B, S, D = x.shape
    NC, NS = 2, 16
    NW = NC * NS
    SPW = S // NW
    CS = 4
    NCH = SPW // CS
    NITER = NCH // 2

    mesh = plsc.VectorSubcoreMesh(core_axis_name="c", subcore_axis_name="s")

    @functools.partial(
        pl.kernel,
        out_type=jax.ShapeDtypeStruct((B, S, D), jnp.float32),
        mesh=mesh,
        scratch_types=[
            pltpu.VMEM((2, CS, D), jnp.float32),
            pltpu.VMEM((B, CS, D), jnp.float32),
            pltpu.SemaphoreType.DMA((2,)),
            pltpu.SemaphoreType.DMA((B,)),
            pltpu.SemaphoreType.DMA((B,)),
        ],
    )
    def sc_add(x_hbm, t_hbm, o_hbm, t_v, xo_v, t_sem, in_sem, out_sem):
        wid = lax.axis_index("s") * NC + lax.axis_index("c")
        s_base = wid * SPW

        def s0(c):
            return s_base + c * CS

        def fire_t(c, p):
            pltpu.async_copy(
                t_hbm.at[pl.ds(s0(c), CS), :], t_v.at[p], t_sem.at[p]
            )

        def wait_t(c, p):
            pltpu.make_async_copy(
                t_hbm.at[pl.ds(s0(c), CS), :], t_v.at[p], t_sem.at[p]
            ).wait()

        def fire_in(c, b):
            pltpu.async_copy(
                x_hbm.at[b, pl.ds(s0(c), CS), :], xo_v.at[b], in_sem.at[b]
            )

        def wait_in(c, b):
            pltpu.make_async_copy(
                x_hbm.at[b, pl.ds(s0(c), CS), :], xo_v.at[b], in_sem.at[b]
            ).wait()

        def fire_out(c, b):
            pltpu.async_copy(
                xo_v.at[b], o_hbm.at[b, pl.ds(s0(c), CS), :], out_sem.at[b]
            )

        def wait_out(c, b):
            pltpu.make_async_copy(
                xo_v.at[b], o_hbm.at[b, pl.ds(s0(c), CS), :], out_sem.at[b]
            ).wait()

        fire_t(0, 0)
        fire_t(1, 1)
        for b in range(B):
            fire_in(0, b)

        @pl.loop(0, NITER)
        def _iter(cc):
            for p in range(2):
                c = 2 * cc + p
                wait_t(c, p)
                for b in range(B):
                    wait_in(c, b)
                    for r in range(CS):

                        @pl.loop(0, D // 16, unroll=16)
                        def _vec(i):
                            sl = pl.ds(i * 16, 16)
                            plsc.addupdate(xo_v.at[b, r, sl], t_v[p, r, sl])

                    fire_out(c, b)

                @pl.when(c + 2 < NCH)
                def _():
                    fire_t(c + 2, p)

                @pl.when(c + 1 < NCH)
                def _():
                    for b in range(B):
                        wait_out(c, b)
                        fire_in(c + 1, b)

        for b in range(B):
            wait_out(NCH - 1, b)

    return sc_add(x, table)


# speedup vs baseline: 1.2766x; 1.2729x over previous
import functools

import jax
import jax.numpy as jnp
from jax import lax
from jax.experimental import pallas as pl
from jax.experimental.pallas import tpu as pltpu
from jax.experimental.pallas import tpu_sc as plsc


def kernel(x, table):
    B, S, D = x.shape
    NC, NS = 2, 16
    NW = NC * NS
    SPW = S // NW
    CS = 4
    NCH = SPW // CS
    NITER = NCH // 2

    mesh = plsc.VectorSubcoreMesh(core_axis_name="c", subcore_axis_name="s")

    @functools.partial(
        pl.kernel,
        out_type=jax.ShapeDtypeStruct((B, S, D), jnp.float32),
        mesh=mesh,
        scratch_types=[
            pltpu.VMEM((2, CS, D), jnp.float32),
            pltpu.VMEM((5, CS, D), jnp.float32),
            pltpu.SemaphoreType.DMA((2,)),
            pltpu.SemaphoreType.DMA((5,)),
            pltpu.SemaphoreType.DMA((5,)),
        ],
    )
    def sc_add(x_hbm, t_hbm, o_hbm, t_v, xo_v, t_sem, in_sem, out_sem):
        wid = lax.axis_index("s") * NC + lax.axis_index("c")
        s_base = wid * SPW

        def s0(c):
            return s_base + c * CS

        def fire_t(c, p):
            pltpu.async_copy(
                t_hbm.at[pl.ds(s0(c), CS), :], t_v.at[p], t_sem.at[p]
            )

        def wait_t(c, p):
            pltpu.make_async_copy(
                t_hbm.at[pl.ds(s0(c), CS), :], t_v.at[p], t_sem.at[p]
            ).wait()

        def fire_in(c, b, m):
            pltpu.async_copy(
                x_hbm.at[b, pl.ds(s0(c), CS), :], xo_v.at[m], in_sem.at[m]
            )

        def wait_in(c, b, m):
            pltpu.make_async_copy(
                x_hbm.at[b, pl.ds(s0(c), CS), :], xo_v.at[m], in_sem.at[m]
            ).wait()

        def fire_out(c, b, m):
            pltpu.async_copy(
                xo_v.at[m], o_hbm.at[b, pl.ds(s0(c), CS), :], out_sem.at[m]
            )

        def wait_out(c, b, m):
            pltpu.make_async_copy(
                xo_v.at[m], o_hbm.at[b, pl.ds(s0(c), CS), :], out_sem.at[m]
            ).wait()

        def compute(m, p):
            for r in range(CS):

                @pl.loop(0, D // 16, unroll=8)
                def _vec(i):
                    sl = pl.ds(i * 16, 16)
                    plsc.addupdate(xo_v.at[m, r, sl], t_v[p, r, sl])

        fire_t(0, 0)
        fire_t(1, 1)
        fire_in(0, 0, 0)
        fire_in(0, 1, 1)
        fire_in(0, 2, 2)

        @pl.loop(0, NITER)
        def _iter(cc):
            for p in range(2):
                c = 2 * cc + p
                m0_cur = 0 if p == 0 else 4
                m0_nxt = 4 if p == 0 else 0
                first = (cc == 0) if p == 0 else None
                last = None if p == 0 else (cc == NITER - 1)

                wait_t(c, p)

                wait_in(c, 0, m0_cur)
                compute(m0_cur, p)
                fire_out(c, 0, m0_cur)

                if first is None:
                    wait_out(c - 1, 3, 3)
                else:

                    @pl.when(jnp.logical_not(first))
                    def _():
                        wait_out(c - 1, 3, 3)

                fire_in(c, 3, 3)

                wait_in(c, 1, 1)
                compute(1, p)
                fire_out(c, 1, 1)

                if first is None:
                    wait_out(c - 1, 0, m0_nxt)
                else:

                    @pl.when(jnp.logical_not(first))
                    def _():
                        wait_out(c - 1, 0, m0_nxt)

                if last is None:
                    fire_in(c + 1, 0, m0_nxt)
                else:

                    @pl.when(jnp.logical_not(last))
                    def _():
                        fire_in(c + 1, 0, m0_nxt)

                wait_in(c, 2, 2)
                compute(2, p)
                fire_out(c, 2, 2)

                wait_out(c, 1, 1)
                if last is None:
                    fire_in(c + 1, 1, 1)
                else:

                    @pl.when(jnp.logical_not(last))
                    def _():
                        fire_in(c + 1, 1, 1)

                wait_in(c, 3, 3)
                compute(3, p)
                fire_out(c, 3, 3)

                @pl.when(cc < NITER - 1)
                def _():
                    fire_t(c + 2, p)

                wait_out(c, 2, 2)
                if last is None:
                    fire_in(c + 1, 2, 2)
                else:

                    @pl.when(jnp.logical_not(last))
                    def _():
                        fire_in(c + 1, 2, 2)

        wait_out(NCH - 1, 0, 4)
        wait_out(NCH - 1, 3, 3)

    return sc_add(x, table)
